# Initial kernel scaffold; baseline (speedup 1.0000x reference)
import functools

import jax
import jax.numpy as jnp
from jax import lax
from jax.experimental import pallas as pl
from jax.experimental.pallas import tpu as pltpu
from jax.experimental.pallas import tpu_sc as plsc

N = 10000
E = 320000
D = 128
G = 64

DP = 144
NC, NS, L = 2, 16, 16
NW = NC * NS
EPT = E // NW
K = 80
NCHUNK = EPT // K
RPT = N // NS
ZCH = 125

_f32 = jnp.float32


def _attn_tail(h, asrc_ref, adst_ref, ht_ref, as_ref, ad_ref):
    ht_ref[...] = jnp.concatenate(
        [h, jnp.ones((N, 1), _f32), jnp.zeros((N, DP - D - 1), _f32)], axis=1)
    as_ref[...] = jnp.sum(h * asrc_ref[...], axis=1, keepdims=True)
    ad_ref[...] = jnp.sum(h * adst_ref[...], axis=1, keepdims=True)


def _head_body(x_ref, w_ref, asrc_ref, adst_ref, ht_ref, as_ref, ad_ref):
    h = jnp.dot(x_ref[...], w_ref[...], preferred_element_type=_f32)
    _attn_tail(h, asrc_ref, adst_ref, ht_ref, as_ref, ad_ref)


def _combine_bn_relu(p_ref, b_ref, g_ref, beta_ref):
    s = p_ref[0] + p_ref[1]
    z = s[:, D:D + 1]
    out = s[:, :D] / (z + 1e-16) + b_ref[...]
    mu = jnp.mean(out, axis=0, keepdims=True)
    var = jnp.mean((out - mu) ** 2, axis=0, keepdims=True)
    y = (out - mu) * lax.rsqrt(var + 1e-5) * g_ref[...] + beta_ref[...]
    return jnp.maximum(y, 0.0)


def _mid_body(p_ref, b_ref, g_ref, beta_ref, w_ref, asrc_ref, adst_ref,
              ht_ref, as_ref, ad_ref):
    y = _combine_bn_relu(p_ref, b_ref, g_ref, beta_ref)
    h = jnp.dot(y, w_ref[...], preferred_element_type=_f32)
    _attn_tail(h, asrc_ref, adst_ref, ht_ref, as_ref, ad_ref)


def _tail_body(p_ref, b_ref, g_ref, beta_ref, batch_ref, l1w_ref, l1b_ref,
               l2w_ref, l2b_ref, out_ref):
    y = _combine_bn_relu(p_ref, b_ref, g_ref, beta_ref)
    gids = lax.broadcasted_iota(jnp.int32, (G, N), 0)
    onehot = (jnp.broadcast_to(batch_ref[...], (G, N)) == gids).astype(_f32)
    sums = jnp.dot(onehot, y, preferred_element_type=_f32)
    cnt = jnp.sum(onehot, axis=1, keepdims=True)
    gfeat = sums / jnp.maximum(cnt, 1.0)
    gfeat = jnp.maximum(
        jnp.dot(gfeat, l1w_ref[...], preferred_element_type=_f32)
        + l1b_ref[...], 0.0)
    out_ref[...] = (jnp.dot(gfeat, l2w_ref[...], preferred_element_type=_f32)
                    + l2b_ref[...])


_head = pl.pallas_call(
    _head_body,
    out_shape=(jax.ShapeDtypeStruct((N, DP), _f32),
               jax.ShapeDtypeStruct((N, 1), _f32),
               jax.ShapeDtypeStruct((N, 1), _f32)))

_mid = pl.pallas_call(
    _mid_body,
    out_shape=(jax.ShapeDtypeStruct((N, DP), _f32),
               jax.ShapeDtypeStruct((N, 1), _f32),
               jax.ShapeDtypeStruct((N, 1), _f32)))

_tail = pl.pallas_call(
    _tail_body,
    out_shape=jax.ShapeDtypeStruct((G, 1), _f32))


def _sc_edge_body(ht_hbm, as_hbm, ad_hbm, src_hbm, dst_hbm, zeros_hbm,
                  out_hbm, asv, adv, srcv, dstv, rowsv, wv, zbuf, acc, sem):
    c = lax.axis_index("c")
    t = lax.axis_index("s")

    pltpu.sync_copy(as_hbm, asv)
    pltpu.sync_copy(ad_hbm, adv)

    pltpu.sync_copy(zeros_hbm, zbuf)
    for r in range(RPT // ZCH):
        pltpu.sync_copy(zbuf, acc.at[pl.ds(t * RPT + r * ZCH, ZCH)])
    plsc.subcore_barrier()

    ebase = (c * NS + t) * EPT

    def chunk(i, carry):
        off = ebase + i * K
        pltpu.sync_copy(src_hbm.at[pl.ds(off, K)], srcv)
        pltpu.sync_copy(dst_hbm.at[pl.ds(off, K)], dstv)
        gat = pltpu.async_copy(ht_hbm.at[srcv], rowsv, sem)

        for j in range(K // L):
            si = srcv[pl.ds(j * L, L)]
            di = dstv[pl.ds(j * L, L)]
            e = plsc.load_gather(asv, [si]) + plsc.load_gather(adv, [di])
            e = jnp.where(e >= 0.0, e, 0.2 * e)
            wv[pl.ds(j * L, L)] = jnp.exp(e)

        gat.wait()

        def scale_row(j, carry2):
            wj = wv[j]
            for v in range(DP // L):
                rowsv[j, pl.ds(v * L, L)] = rowsv[j, pl.ds(v * L, L)] * wj
            return carry2
        lax.fori_loop(0, K, scale_row, 0)

        pltpu.sync_copy(rowsv, acc.at[dstv], add=True)
        return carry

    lax.fori_loop(0, NCHUNK, chunk, 0)
    plsc.subcore_barrier()

    for r in range(RPT // ZCH):
        base = t * RPT + r * ZCH
        pltpu.sync_copy(acc.at[pl.ds(base, ZCH)], zbuf)
        pltpu.sync_copy(zbuf, out_hbm.at[c, pl.ds(base, ZCH)])


_sc_edge = pl.kernel(
    _sc_edge_body,
    out_type=jax.ShapeDtypeStruct((NC, N, DP), _f32),
    mesh=plsc.VectorSubcoreMesh(core_axis_name="c", subcore_axis_name="s"),
    scratch_types=[
        pltpu.VMEM((N,), _f32),
        pltpu.VMEM((N,), _f32),
        pltpu.VMEM((K,), jnp.int32),
        pltpu.VMEM((K,), jnp.int32),
        pltpu.VMEM((K, DP), _f32),
        pltpu.VMEM((K,), _f32),
        pltpu.VMEM((ZCH, DP), _f32),
        pltpu.VMEM_SHARED((N, DP), _f32),
        pltpu.SemaphoreType.DMA,
    ])


def kernel(x, edge_index, batch, params):
    src = edge_index[0]
    dst = edge_index[1]
    zeros = jnp.zeros((ZCH, DP), _f32)

    p1, p2, p3 = params["gat1"], params["gat2"], params["gat3"]
    bn1, bn2, bn3 = params["bn1"], params["bn2"], params["bn3"]

    ht, a_s, a_d = _head(x, p1["W"], p1["a_src"], p1["a_dst"])
    part = _sc_edge(ht, a_s.reshape(N), a_d.reshape(N), src, dst, zeros)
    ht, a_s, a_d = _mid(part, p1["b"], bn1["g"], bn1["b"],
                        p2["W"], p2["a_src"], p2["a_dst"])
    part = _sc_edge(ht, a_s.reshape(N), a_d.reshape(N), src, dst, zeros)
    ht, a_s, a_d = _mid(part, p2["b"], bn2["g"], bn2["b"],
                        p3["W"], p3["a_src"], p3["a_dst"])
    part = _sc_edge(ht, a_s.reshape(N), a_d.reshape(N), src, dst, zeros)
    return _tail(part, p3["b"], bn3["g"], bn3["b"], batch.reshape(1, N),
                 params["lin1_W"], params["lin1_b"],
                 params["lin2_W"], params["lin2_b"])


# speedup vs baseline: 22.2477x; 22.2477x over previous
import functools

import jax
import jax.numpy as jnp
from jax import lax
from jax.experimental import pallas as pl
from jax.experimental.pallas import tpu as pltpu
from jax.experimental.pallas import tpu_sc as plsc

N = 10000
E = 320000
D = 128
G = 64

DP = 144
NC, NS, L = 2, 16, 16
NW = NC * NS
EPT = E // NW
K = 80
NCHUNK = EPT // K
NP = 10240
RPT = NP // NS
ZCH = 40

_f32 = jnp.float32


def _attn_tail(h, asrc_ref, adst_ref, ht_ref, as_ref, ad_ref):
    ht_ref[...] = jnp.concatenate(
        [h, jnp.ones((N, 1), _f32), jnp.zeros((N, DP - D - 1), _f32)], axis=1)
    as_ref[...] = jnp.sum(h * asrc_ref[...], axis=1, keepdims=True)
    ad_ref[...] = jnp.sum(h * adst_ref[...], axis=1, keepdims=True)


def _head_body(x_ref, w_ref, asrc_ref, adst_ref, ht_ref, as_ref, ad_ref):
    h = jnp.dot(x_ref[...], w_ref[...], preferred_element_type=_f32,
                precision=lax.Precision.HIGHEST)
    _attn_tail(h, asrc_ref, adst_ref, ht_ref, as_ref, ad_ref)


def _combine_bn_relu(p_ref, b_ref, g_ref, beta_ref):
    s = p_ref[0, :N] + p_ref[1, :N]
    z = s[:, D:D + 1]
    out = s[:, :D] / (z + 1e-16) + b_ref[...]
    mu = jnp.mean(out, axis=0, keepdims=True)
    var = jnp.mean((out - mu) ** 2, axis=0, keepdims=True)
    y = (out - mu) * lax.rsqrt(var + 1e-5) * g_ref[...] + beta_ref[...]
    return jnp.maximum(y, 0.0)


def _mid_body(p_ref, b_ref, g_ref, beta_ref, w_ref, asrc_ref, adst_ref,
              ht_ref, as_ref, ad_ref):
    y = _combine_bn_relu(p_ref, b_ref, g_ref, beta_ref)
    h = jnp.dot(y, w_ref[...], preferred_element_type=_f32,
                precision=lax.Precision.HIGHEST)
    _attn_tail(h, asrc_ref, adst_ref, ht_ref, as_ref, ad_ref)


def _tail_body(p_ref, b_ref, g_ref, beta_ref, batch_ref, l1w_ref, l1b_ref,
               l2w_ref, l2b_ref, out_ref):
    y = _combine_bn_relu(p_ref, b_ref, g_ref, beta_ref)
    gids = lax.broadcasted_iota(jnp.int32, (G, N), 0)
    onehot = (jnp.broadcast_to(batch_ref[...], (G, N)) == gids).astype(_f32)
    sums = jnp.dot(onehot, y, preferred_element_type=_f32,
                precision=lax.Precision.HIGHEST)
    cnt = jnp.sum(onehot, axis=1, keepdims=True)
    gfeat = sums / jnp.maximum(cnt, 1.0)
    gfeat = jnp.maximum(
        jnp.dot(gfeat, l1w_ref[...], preferred_element_type=_f32,
                precision=lax.Precision.HIGHEST)
        + l1b_ref[...], 0.0)
    out_ref[...] = (jnp.dot(gfeat, l2w_ref[...], preferred_element_type=_f32,
                precision=lax.Precision.HIGHEST)
                    + l2b_ref[...])


_tc_params = pltpu.CompilerParams(vmem_limit_bytes=100 * 1024 * 1024)

_head = pl.pallas_call(
    _head_body,
    out_shape=(jax.ShapeDtypeStruct((N, DP), _f32),
               jax.ShapeDtypeStruct((N, 1), _f32),
               jax.ShapeDtypeStruct((N, 1), _f32)),
    compiler_params=_tc_params)

_mid = pl.pallas_call(
    _mid_body,
    out_shape=(jax.ShapeDtypeStruct((N, DP), _f32),
               jax.ShapeDtypeStruct((N, 1), _f32),
               jax.ShapeDtypeStruct((N, 1), _f32)),
    compiler_params=_tc_params)

_tail = pl.pallas_call(
    _tail_body,
    out_shape=jax.ShapeDtypeStruct((G, 1), _f32),
    compiler_params=_tc_params)


def _sc_edge_body(ht_hbm, as_hbm, ad_hbm, src_hbm, dst_hbm, zeros_hbm,
                  out_hbm, asv, adv, srcv, dstv, rowsv, wv, zbuf, acc, sem):
    c = lax.axis_index("c")
    t = lax.axis_index("s")

    pltpu.sync_copy(as_hbm, asv)
    pltpu.sync_copy(ad_hbm, adv)

    pltpu.sync_copy(zeros_hbm, zbuf)
    for r in range(RPT // ZCH):
        pltpu.sync_copy(zbuf, acc.at[pl.ds(t * RPT + r * ZCH, ZCH)])
    plsc.subcore_barrier()

    ebase = (c * NS + t) * EPT

    def chunk(i, carry):
        off = ebase + i * K
        pltpu.sync_copy(src_hbm.at[pl.ds(off, K)], srcv)
        pltpu.sync_copy(dst_hbm.at[pl.ds(off, K)], dstv)
        gat = pltpu.async_copy(ht_hbm.at[srcv], rowsv, sem)

        for j in range(K // L):
            si = srcv[pl.ds(j * L, L)]
            di = dstv[pl.ds(j * L, L)]
            e = plsc.load_gather(asv, [si]) + plsc.load_gather(adv, [di])
            e = jnp.where(e >= 0.0, e, 0.2 * e)
            wv[pl.ds(j * L, L)] = jnp.exp(e)

        gat.wait()

        def scale_grp(jg, carry2):
            wvec = wv[pl.ds(jg * L, L)]
            for jj in range(L):
                wj = wvec[jj]
                row = jg * L + jj
                for v in range(DP // L):
                    sl = pl.ds(v * L, L)
                    rowsv[row, sl] = rowsv[row, sl] * wj
            return carry2
        lax.fori_loop(0, K // L, scale_grp, 0)

        pltpu.sync_copy(rowsv, acc.at[dstv], add=True)
        return carry

    lax.fori_loop(0, NCHUNK, chunk, 0)
    plsc.subcore_barrier()

    for r in range(RPT // ZCH):
        base = t * RPT + r * ZCH
        pltpu.sync_copy(acc.at[pl.ds(base, ZCH)], zbuf)
        pltpu.sync_copy(zbuf, out_hbm.at[c, pl.ds(base, ZCH)])


_sc_edge = pl.kernel(
    _sc_edge_body,
    out_type=jax.ShapeDtypeStruct((NC, NP, DP), _f32),
    mesh=plsc.VectorSubcoreMesh(core_axis_name="c", subcore_axis_name="s"),
    compiler_params=pltpu.CompilerParams(use_tc_tiling_on_sc=False,
                                         needs_layout_passes=False),
    scratch_types=[
        pltpu.VMEM((N,), _f32),
        pltpu.VMEM((N,), _f32),
        pltpu.VMEM((K,), jnp.int32),
        pltpu.VMEM((K,), jnp.int32),
        pltpu.VMEM((K, DP), _f32),
        pltpu.VMEM((K,), _f32),
        pltpu.VMEM((ZCH, DP), _f32),
        pltpu.VMEM_SHARED((NP, DP), _f32),
        pltpu.SemaphoreType.DMA,
    ])


def kernel(x, edge_index, batch, params):
    src = edge_index[0]
    dst = edge_index[1]
    zeros = jnp.zeros((ZCH, DP), _f32)

    p1, p2, p3 = params["gat1"], params["gat2"], params["gat3"]
    bn1, bn2, bn3 = params["bn1"], params["bn2"], params["bn3"]

    ht, a_s, a_d = _head(x, p1["W"], p1["a_src"], p1["a_dst"])
    part = _sc_edge(ht, a_s.reshape(N), a_d.reshape(N), src, dst, zeros)
    ht, a_s, a_d = _mid(part, p1["b"], bn1["g"], bn1["b"],
                        p2["W"], p2["a_src"], p2["a_dst"])
    part = _sc_edge(ht, a_s.reshape(N), a_d.reshape(N), src, dst, zeros)
    ht, a_s, a_d = _mid(part, p2["b"], bn2["g"], bn2["b"],
                        p3["W"], p3["a_src"], p3["a_dst"])
    part = _sc_edge(ht, a_s.reshape(N), a_d.reshape(N), src, dst, zeros)
    return _tail(part, p3["b"], bn3["g"], bn3["b"], batch.reshape(1, N),
                 params["lin1_W"], params["lin1_b"],
                 params["lin2_W"], params["lin2_b"])


# speedup vs baseline: 31.3984x; 1.4113x over previous
import functools

import jax
import jax.numpy as jnp
from jax import lax
from jax.experimental import pallas as pl
from jax.experimental.pallas import tpu as pltpu
from jax.experimental.pallas import tpu_sc as plsc

N = 10000
E = 320000
D = 128
G = 64

DP = 144
NC, NS, L = 2, 16, 16
NW = NC * NS
EPT = E // NW
K = 80
NCHUNK = EPT // K
NP = 10240
RPT = NP // NS

_f32 = jnp.float32


def _attn_tail(h, asrc_ref, adst_ref, ht_ref, ad_ref):
    a_s = jnp.sum(h * asrc_ref[...], axis=1, keepdims=True)
    ht_ref[...] = jnp.concatenate(
        [h, jnp.ones((N, 1), _f32), a_s, jnp.zeros((N, DP - D - 2), _f32)],
        axis=1)
    ad_ref[...] = jnp.sum(h * adst_ref[...], axis=1, keepdims=True)


def _head_body(x_ref, w_ref, asrc_ref, adst_ref, ht_ref, ad_ref):
    h = jnp.dot(x_ref[...], w_ref[...], preferred_element_type=_f32,
                precision=lax.Precision.HIGHEST)
    _attn_tail(h, asrc_ref, adst_ref, ht_ref, ad_ref)


def _combine_bn_relu(p_ref, b_ref, g_ref, beta_ref):
    s = p_ref[0, :N] + p_ref[1, :N]
    z = s[:, D:D + 1]
    out = s[:, :D] / (z + 1e-16) + b_ref[...]
    mu = jnp.mean(out, axis=0, keepdims=True)
    var = jnp.mean((out - mu) ** 2, axis=0, keepdims=True)
    y = (out - mu) * lax.rsqrt(var + 1e-5) * g_ref[...] + beta_ref[...]
    return jnp.maximum(y, 0.0)


def _mid_body(p_ref, b_ref, g_ref, beta_ref, w_ref, asrc_ref, adst_ref,
              ht_ref, ad_ref):
    y = _combine_bn_relu(p_ref, b_ref, g_ref, beta_ref)
    h = jnp.dot(y, w_ref[...], preferred_element_type=_f32,
                precision=lax.Precision.HIGHEST)
    _attn_tail(h, asrc_ref, adst_ref, ht_ref, ad_ref)


def _tail_body(p_ref, b_ref, g_ref, beta_ref, batch_ref, l1w_ref, l1b_ref,
               l2w_ref, l2b_ref, out_ref):
    y = _combine_bn_relu(p_ref, b_ref, g_ref, beta_ref)
    gids = lax.broadcasted_iota(jnp.int32, (G, N), 0)
    onehot = (jnp.broadcast_to(batch_ref[...], (G, N)) == gids).astype(_f32)
    sums = jnp.dot(onehot, y, preferred_element_type=_f32,
                   precision=lax.Precision.HIGHEST)
    cnt = jnp.sum(onehot, axis=1, keepdims=True)
    gfeat = sums / jnp.maximum(cnt, 1.0)
    gfeat = jnp.maximum(
        jnp.dot(gfeat, l1w_ref[...], preferred_element_type=_f32,
                precision=lax.Precision.HIGHEST)
        + l1b_ref[...], 0.0)
    out_ref[...] = (jnp.dot(gfeat, l2w_ref[...], preferred_element_type=_f32,
                            precision=lax.Precision.HIGHEST)
                    + l2b_ref[...])


_tc_params = pltpu.CompilerParams(vmem_limit_bytes=100 * 1024 * 1024)

_head = pl.pallas_call(
    _head_body,
    out_shape=(jax.ShapeDtypeStruct((N, DP), _f32),
               jax.ShapeDtypeStruct((N, 1), _f32)),
    compiler_params=_tc_params)

_mid = pl.pallas_call(
    _mid_body,
    out_shape=(jax.ShapeDtypeStruct((N, DP), _f32),
               jax.ShapeDtypeStruct((N, 1), _f32)),
    compiler_params=_tc_params)

_tail = pl.pallas_call(
    _tail_body,
    out_shape=jax.ShapeDtypeStruct((G, 1), _f32),
    compiler_params=_tc_params)


def _sc_edge_body(ht_hbm, ad_hbm, src_hbm, dst_hbm, zeros_hbm, out_hbm,
                  adv, s0, d0, s1, d1, r0, r1, acc, g0, g1, c0, c1):
    cid = lax.axis_index("c")
    t = lax.axis_index("s")
    S = (s0, s1)
    Dd = (d0, d1)
    R = (r0, r1)
    Gs = (g0, g1)
    Cs = (c0, c1)

    pltpu.sync_copy(ad_hbm, adv)

    pltpu.sync_copy(zeros_hbm, r0)
    for r in range(RPT // K):
        pltpu.sync_copy(r0, acc.at[pl.ds(t * RPT + r * K, K)])
    plsc.subcore_barrier()

    ebase = (cid * NS + t) * EPT

    def fetch(ib, b):
        off = ebase + ib * K
        pltpu.sync_copy(src_hbm.at[pl.ds(off, K)], S[b])
        pltpu.sync_copy(dst_hbm.at[pl.ds(off, K)], Dd[b])
        pltpu.async_copy(ht_hbm.at[S[b]], R[b], Gs[b])

    def step(ib, b):
        nb = 1 - b

        @pl.when(ib + 1 < NCHUNK)
        def _():
            fetch(ib + 1, nb)

        pltpu.make_async_copy(ht_hbm.at[S[b]], R[b], Gs[b]).wait()

        rows = R[b]
        dstv = Dd[b]

        def grp(jg, carry):
            di = dstv[pl.ds(jg * L, L)]
            advec = plsc.load_gather(adv, [di])
            rowid = jg * L + lax.iota(jnp.int32, L)
            asvec = plsc.load_gather(
                rows, [rowid, jnp.full((L,), D + 1, jnp.int32)])
            e = asvec + advec
            e = jnp.where(e >= 0.0, e, 0.2 * e)
            wvec = jnp.exp(e)
            for jj in range(L):
                wj = wvec[jj]
                row = jg * L + jj
                for v in range(DP // L):
                    sl = pl.ds(v * L, L)
                    rows[row, sl] = rows[row, sl] * wj
            return carry
        lax.fori_loop(0, K // L, grp, 0)

        pltpu.sync_copy(rows, acc.at[dstv], add=True)

    fetch(0, 0)

    def pair(ip, carry):
        step(2 * ip, 0)
        step(2 * ip + 1, 1)
        return carry
    lax.fori_loop(0, NCHUNK // 2, pair, 0)
    step(NCHUNK - 1, 0)

    plsc.subcore_barrier()

    for r in range(RPT // K):
        base = t * RPT + r * K
        pltpu.sync_copy(acc.at[pl.ds(base, K)], r0)
        pltpu.sync_copy(r0, out_hbm.at[cid, pl.ds(base, K)])


_sc_edge = pl.kernel(
    _sc_edge_body,
    out_type=jax.ShapeDtypeStruct((NC, NP, DP), _f32),
    mesh=plsc.VectorSubcoreMesh(core_axis_name="c", subcore_axis_name="s"),
    compiler_params=pltpu.CompilerParams(use_tc_tiling_on_sc=False,
                                         needs_layout_passes=False),
    scratch_types=[
        pltpu.VMEM((N,), _f32),
        pltpu.VMEM((K,), jnp.int32),
        pltpu.VMEM((K,), jnp.int32),
        pltpu.VMEM((K,), jnp.int32),
        pltpu.VMEM((K,), jnp.int32),
        pltpu.VMEM((K, DP), _f32),
        pltpu.VMEM((K, DP), _f32),
        pltpu.VMEM_SHARED((NP, DP), _f32),
        pltpu.SemaphoreType.DMA,
        pltpu.SemaphoreType.DMA,
        pltpu.SemaphoreType.DMA,
        pltpu.SemaphoreType.DMA,
    ])


def kernel(x, edge_index, batch, params):
    src = edge_index[0]
    dst = edge_index[1]
    zeros = jnp.zeros((K, DP), _f32)

    p1, p2, p3 = params["gat1"], params["gat2"], params["gat3"]
    bn1, bn2, bn3 = params["bn1"], params["bn2"], params["bn3"]

    ht, a_d = _head(x, p1["W"], p1["a_src"], p1["a_dst"])
    part = _sc_edge(ht, a_d.reshape(N), src, dst, zeros)
    ht, a_d = _mid(part, p1["b"], bn1["g"], bn1["b"],
                   p2["W"], p2["a_src"], p2["a_dst"])
    part = _sc_edge(ht, a_d.reshape(N), src, dst, zeros)
    ht, a_d = _mid(part, p2["b"], bn2["g"], bn2["b"],
                   p3["W"], p3["a_src"], p3["a_dst"])
    part = _sc_edge(ht, a_d.reshape(N), src, dst, zeros)
    return _tail(part, p3["b"], bn3["g"], bn3["b"], batch.reshape(1, N),
                 params["lin1_W"], params["lin1_b"],
                 params["lin2_W"], params["lin2_b"])
